# baseline (speedup 1.0000x reference)
import functools

import jax
import jax.numpy as jnp
from jax import lax
from jax.experimental import pallas as pl
from jax.experimental.pallas import tpu as pltpu
from jax.experimental.pallas import tpu_sc as plsc

N_FIELDS = 26
VOCAB = 40000
EMBED = 200
BATCH = 4096
HID = 200

NC = 2
NS = 16
NW = NC * NS
ROWS = BATCH * N_FIELDS
CHUNK = 128
ROWS_PER_W = ROWS // NW
CHUNKS_PER_W = ROWS_PER_W // CHUNK

VB = 4096
NB = -(-VOCAB // VB)


def _round_bf16_hi(x):
    b = lax.bitcast_convert_type(x, jnp.int32)
    lsb = lax.shift_right_logical(b, 16) & 1
    return (b + 0x7FFF + lsb) & jnp.int32(-65536)


def _tr_body(t_ref, p_ref):
    y = t_ref[...].T
    hi = _round_bf16_hi(y[:, :128])
    lo = _round_bf16_hi(jnp.concatenate(
        [y[:, 128:], jnp.zeros((VB, 128 - (EMBED - 128)), jnp.float32)],
        axis=1))
    packed = hi | lax.shift_right_logical(lo, 16)
    p_ref[...] = lax.bitcast_convert_type(packed, jnp.float32)[None]


FGROUPS = (7, 7, 6, 6)


def _make_tr_call(f0, nf):
    return pl.pallas_call(
        _tr_body,
        grid=(nf, NB),
        in_specs=[pl.BlockSpec((EMBED, VB), lambda i, j: (i + f0, j))],
        out_specs=pl.BlockSpec((1, VB, 128), lambda i, j: (i, j, 0)),
        out_shape=jax.ShapeDtypeStruct((nf, VOCAB, 128), jnp.float32),
    )


@functools.lru_cache(maxsize=None)
def _make_sc_gather(n_rows):
    mesh = plsc.VectorSubcoreMesh(core_axis_name="c", subcore_axis_name="s")
    rows_per_w = n_rows // NW
    n_chunks = rows_per_w // CHUNK

    @functools.partial(
        pl.kernel,
        mesh=mesh,
        out_type=jax.ShapeDtypeStruct((n_rows, 128), jnp.float32),
        scratch_types=[
            pltpu.VMEM((n_chunks, CHUNK), jnp.int32),
            pltpu.VMEM((CHUNK, 128), jnp.float32),
            pltpu.SemaphoreType.DMA,
        ],
    )
    def _sc_gather(idx_hbm, tabp_hbm, embp_out, idx_v, pbuf, psem):
        wid = lax.axis_index("s") * NC + lax.axis_index("c")
        pltpu.sync_copy(idx_hbm.at[wid], idx_v)

        def chunk(j, carry):
            row0 = wid * rows_per_w + j * CHUNK
            pltpu.async_copy(tabp_hbm.at[idx_v.at[j]], pbuf, psem).wait()
            pltpu.sync_copy(pbuf, embp_out.at[pl.ds(row0, CHUNK)])
            return carry

        lax.fori_loop(0, n_chunks, chunk, 0)

    return _sc_gather


@functools.lru_cache(maxsize=None)
def _make_sc_wgather():
    mesh = plsc.VectorSubcoreMesh(core_axis_name="c", subcore_axis_name="s")

    @functools.partial(
        pl.kernel,
        mesh=mesh,
        out_type=jax.ShapeDtypeStruct((ROWS, 16), jnp.float32),
        scratch_types=[
            pltpu.VMEM((CHUNKS_PER_W, CHUNK), jnp.int32),
            pltpu.VMEM((CHUNK, 16), jnp.float32),
            pltpu.SemaphoreType.DMA,
        ],
        compiler_params=pltpu.CompilerParams(use_tc_tiling_on_sc=False),
    )
    def _sc_wgather(idxw_hbm, w16_hbm, w16_out, idxw_v, wbuf, wsem):
        wid = lax.axis_index("s") * NC + lax.axis_index("c")
        pltpu.sync_copy(idxw_hbm.at[wid], idxw_v)

        def chunk(j, carry):
            row0 = wid * ROWS_PER_W + j * CHUNK
            pltpu.async_copy(w16_hbm.at[idxw_v.at[j]], wbuf, wsem).wait()
            pltpu.sync_copy(wbuf, w16_out.at[pl.ds(row0, CHUNK)])
            return carry

        lax.fori_loop(0, CHUNKS_PER_W, chunk, 0)

    return _sc_wgather


BB = 256
DA = N_FIELDS * 128
_F0 = (0, 7, 14, 20)


def _tc_body(xp1_ref, xp2_ref, xp3_ref, xp4_ref, wv_ref, lo_ref,
             w1a_ref, w1b_ref, b1_ref, w2_ref, b2_ref,
             w3_ref, b3_ref, wo_ref, bo_ref, o_ref):
    lo = lo_ref[...]
    xp_refs = (xp1_ref, xp2_ref, xp3_ref, xp4_ref)
    iota = lax.broadcasted_iota(jnp.int32, (BB, 16), 1)
    sa = jnp.zeros((BB, 128), jnp.float32)
    sb = jnp.zeros((BB, 128), jnp.float32)
    sq = jnp.zeros((BB, 1), jnp.float32)
    first = jnp.zeros((BB, 1), jnp.float32)
    h = jnp.zeros((BB, HID), jnp.float32)
    for f in range(N_FIELDS):
        c0, c1 = f * 128, (f + 1) * 128
        g = sum(1 for s in _F0[1:] if f >= s)
        ref = xp_refs[g]
        d0 = (f - _F0[g]) * 128
        p = lax.bitcast_convert_type(ref[:, d0:d0 + 128], jnp.int32)
        xa = lax.bitcast_convert_type(p & jnp.int32(-65536), jnp.float32)
        xb = lax.bitcast_convert_type(lax.shift_left(p, 16), jnp.float32)
        sa = sa + xa
        sb = sb + xb
        sq = sq + jnp.sum(xa * xa + xb * xb, axis=1, keepdims=True)
        ohf = (lo[:, f:f + 1] == iota).astype(jnp.float32)
        first = first + jnp.sum(wv_ref[:, f * 16:(f + 1) * 16] * ohf,
                                axis=1, keepdims=True)
        h = h + jnp.dot(xa.astype(jnp.bfloat16), w1a_ref[c0:c1, :],
                        preferred_element_type=jnp.float32)
        h = h + jnp.dot(xb.astype(jnp.bfloat16), w1b_ref[c0:c1, :],
                        preferred_element_type=jnp.float32)
    second = 0.5 * (jnp.sum(sa * sa, axis=1, keepdims=True)
                    + jnp.sum(sb * sb, axis=1, keepdims=True) - sq)
    h = jax.nn.relu(h + b1_ref[...])
    h = jax.nn.relu(jnp.dot(h.astype(jnp.bfloat16), w2_ref[...],
                            preferred_element_type=jnp.float32) + b2_ref[...])
    h = jax.nn.relu(jnp.dot(h.astype(jnp.bfloat16), w3_ref[...],
                            preferred_element_type=jnp.float32) + b3_ref[...])
    deep = (jnp.dot(h.astype(jnp.bfloat16), wo_ref[...],
                    preferred_element_type=jnp.float32) + bo_ref[...])
    o_ref[...] = jax.nn.sigmoid(first + second + deep)


def _full(shape):
    return pl.BlockSpec(shape, lambda i: (0,) * len(shape))


_tc_call = pl.pallas_call(
    _tc_body,
    grid=(BATCH // BB,),
    in_specs=[
        pl.BlockSpec((BB, FGROUPS[0] * 128), lambda i: (i, 0)),
        pl.BlockSpec((BB, FGROUPS[1] * 128), lambda i: (i, 0)),
        pl.BlockSpec((BB, FGROUPS[2] * 128), lambda i: (i, 0)),
        pl.BlockSpec((BB, FGROUPS[3] * 128), lambda i: (i, 0)),
        pl.BlockSpec((BB, N_FIELDS * 16), lambda i: (i, 0)),
        pl.BlockSpec((BB, N_FIELDS), lambda i: (i, 0)),
        _full((DA, HID)),
        _full((DA, HID)),
        _full((1, HID)),
        _full((HID, HID)),
        _full((1, HID)),
        _full((HID, HID)),
        _full((1, HID)),
        _full((HID, 1)),
        _full((1, 1)),
    ],
    out_specs=pl.BlockSpec((BB, 1), lambda i: (i, 0)),
    out_shape=jax.ShapeDtypeStruct((BATCH, 1), jnp.float32),
)


def kernel(sparse_inputs, tables, w, W1, b1, W2, b2, W3, b3, Wout, bout):
    tt = jnp.transpose(tables, (0, 2, 1)).reshape(N_FIELDS * EMBED, VOCAB)

    offs = (jnp.arange(N_FIELDS, dtype=jnp.int32) * VOCAB)[None, :]
    idx = (sparse_inputs.astype(jnp.int32) + offs)
    w16 = w.reshape(N_FIELDS * VOCAB // 16, 16)
    w16rows = _make_sc_wgather()((idx >> 4).reshape(NW, CHUNKS_PER_W, CHUNK), w16)

    xps = []
    f0 = 0
    for nf in FGROUPS:
        tabp = _make_tr_call(f0, nf)(tt).reshape(nf * VOCAB, 128)
        g_rows = BATCH * nf
        idxh = (idx[:, f0:f0 + nf] - f0 * VOCAB)
        idxh = idxh.reshape(NW, g_rows // (NW * CHUNK), CHUNK)
        xps.append(_make_sc_gather(g_rows)(idxh, tabp).reshape(BATCH, nf * 128))
        f0 += nf

    wv = w16rows.reshape(BATCH, N_FIELDS * 16)
    lo = (idx & 15).reshape(BATCH, N_FIELDS)

    w1s = W1.reshape(N_FIELDS, EMBED, HID)
    w1a = w1s[:, :128, :].reshape(DA, HID).astype(jnp.bfloat16)
    w1b = jnp.pad(w1s[:, 128:, :], ((0, 0), (0, 128 - (EMBED - 128)), (0, 0))
                  ).reshape(DA, HID).astype(jnp.bfloat16)

    return _tc_call(xps[0], xps[1], xps[2], xps[3], wv, lo, w1a, w1b,
                    b1.reshape(1, HID), W2.astype(jnp.bfloat16),
                    b2.reshape(1, HID), W3.astype(jnp.bfloat16),
                    b3.reshape(1, HID), Wout.astype(jnp.bfloat16),
                    bout.reshape(1, 1))


# speedup vs baseline: 1.0231x; 1.0231x over previous
import functools

import jax
import jax.numpy as jnp
from jax import lax
from jax.experimental import pallas as pl
from jax.experimental.pallas import tpu as pltpu
from jax.experimental.pallas import tpu_sc as plsc

N_FIELDS = 26
VOCAB = 40000
EMBED = 200
BATCH = 4096
HID = 200

NC = 2
NS = 16
NW = NC * NS
ROWS = BATCH * N_FIELDS
CHUNK = 128
ROWS_PER_W = ROWS // NW
CHUNKS_PER_W = ROWS_PER_W // CHUNK

VB = 4096
NB = -(-VOCAB // VB)


def _round_bf16_hi(x):
    b = lax.bitcast_convert_type(x, jnp.int32)
    lsb = lax.shift_right_logical(b, 16) & 1
    return (b + 0x7FFF + lsb) & jnp.int32(-65536)


def _tr_body(t_ref, p_ref):
    y = t_ref[...].T
    hi = _round_bf16_hi(y[:, :128])
    lo = _round_bf16_hi(jnp.concatenate(
        [y[:, 128:], jnp.zeros((VB, 128 - (EMBED - 128)), jnp.float32)],
        axis=1))
    packed = hi | lax.shift_right_logical(lo, 16)
    p_ref[...] = lax.bitcast_convert_type(packed, jnp.float32)[None]


FGROUPS = (13, 13)


def _make_tr_call(f0, nf):
    return pl.pallas_call(
        _tr_body,
        grid=(nf, NB),
        in_specs=[pl.BlockSpec((EMBED, VB), lambda i, j: (i + f0, j))],
        out_specs=pl.BlockSpec((1, VB, 128), lambda i, j: (i, j, 0)),
        out_shape=jax.ShapeDtypeStruct((nf, VOCAB, 128), jnp.float32),
    )


@functools.lru_cache(maxsize=None)
def _make_sc_gather(n_rows):
    mesh = plsc.VectorSubcoreMesh(core_axis_name="c", subcore_axis_name="s")
    rows_per_w = n_rows // NW
    n_chunks = rows_per_w // CHUNK

    @functools.partial(
        pl.kernel,
        mesh=mesh,
        out_type=jax.ShapeDtypeStruct((n_rows, 128), jnp.float32),
        scratch_types=[
            pltpu.VMEM((n_chunks, CHUNK), jnp.int32),
            pltpu.VMEM((CHUNK, 128), jnp.float32),
            pltpu.SemaphoreType.DMA,
        ],
    )
    def _sc_gather(idx_hbm, tabp_hbm, embp_out, idx_v, pbuf, psem):
        wid = lax.axis_index("s") * NC + lax.axis_index("c")
        pltpu.sync_copy(idx_hbm.at[wid], idx_v)

        def chunk(j, carry):
            row0 = wid * rows_per_w + j * CHUNK
            pltpu.async_copy(tabp_hbm.at[idx_v.at[j]], pbuf, psem).wait()
            pltpu.sync_copy(pbuf, embp_out.at[pl.ds(row0, CHUNK)])
            return carry

        lax.fori_loop(0, n_chunks, chunk, 0)

    return _sc_gather


@functools.lru_cache(maxsize=None)
def _make_sc_wgather():
    mesh = plsc.VectorSubcoreMesh(core_axis_name="c", subcore_axis_name="s")

    @functools.partial(
        pl.kernel,
        mesh=mesh,
        out_type=jax.ShapeDtypeStruct((ROWS, 16), jnp.float32),
        scratch_types=[
            pltpu.VMEM((CHUNKS_PER_W, CHUNK), jnp.int32),
            pltpu.VMEM((CHUNK, 16), jnp.float32),
            pltpu.SemaphoreType.DMA,
        ],
        compiler_params=pltpu.CompilerParams(use_tc_tiling_on_sc=False),
    )
    def _sc_wgather(idxw_hbm, w16_hbm, w16_out, idxw_v, wbuf, wsem):
        wid = lax.axis_index("s") * NC + lax.axis_index("c")
        pltpu.sync_copy(idxw_hbm.at[wid], idxw_v)

        def chunk(j, carry):
            row0 = wid * ROWS_PER_W + j * CHUNK
            pltpu.async_copy(w16_hbm.at[idxw_v.at[j]], wbuf, wsem).wait()
            pltpu.sync_copy(wbuf, w16_out.at[pl.ds(row0, CHUNK)])
            return carry

        lax.fori_loop(0, CHUNKS_PER_W, chunk, 0)

    return _sc_wgather


BB = 256
DA = N_FIELDS * 128
_F0 = (0, 13)


def _tc_body(xp1_ref, xp2_ref, wv_ref, lo_ref,
             w1a_ref, w1b_ref, b1_ref, w2_ref, b2_ref,
             w3_ref, b3_ref, wo_ref, bo_ref, o_ref):
    lo = lo_ref[...]
    xp_refs = (xp1_ref, xp2_ref)
    iota = lax.broadcasted_iota(jnp.int32, (BB, 16), 1)
    sa = jnp.zeros((BB, 128), jnp.float32)
    sb = jnp.zeros((BB, 128), jnp.float32)
    sq = jnp.zeros((BB, 1), jnp.float32)
    first = jnp.zeros((BB, 1), jnp.float32)
    h = jnp.zeros((BB, HID), jnp.float32)
    for f in range(N_FIELDS):
        c0, c1 = f * 128, (f + 1) * 128
        g = sum(1 for s in _F0[1:] if f >= s)
        ref = xp_refs[g]
        d0 = (f - _F0[g]) * 128
        p = lax.bitcast_convert_type(ref[:, d0:d0 + 128], jnp.int32)
        xa = lax.bitcast_convert_type(p & jnp.int32(-65536), jnp.float32)
        xb = lax.bitcast_convert_type(lax.shift_left(p, 16), jnp.float32)
        sa = sa + xa
        sb = sb + xb
        sq = sq + jnp.sum(xa * xa + xb * xb, axis=1, keepdims=True)
        ohf = (lo[:, f:f + 1] == iota).astype(jnp.float32)
        first = first + jnp.sum(wv_ref[:, f * 16:(f + 1) * 16] * ohf,
                                axis=1, keepdims=True)
        h = h + jnp.dot(xa.astype(jnp.bfloat16), w1a_ref[c0:c1, :],
                        preferred_element_type=jnp.float32)
        h = h + jnp.dot(xb.astype(jnp.bfloat16), w1b_ref[c0:c1, :],
                        preferred_element_type=jnp.float32)
    second = 0.5 * (jnp.sum(sa * sa, axis=1, keepdims=True)
                    + jnp.sum(sb * sb, axis=1, keepdims=True) - sq)
    h = jax.nn.relu(h + b1_ref[...])
    h = jax.nn.relu(jnp.dot(h.astype(jnp.bfloat16), w2_ref[...],
                            preferred_element_type=jnp.float32) + b2_ref[...])
    h = jax.nn.relu(jnp.dot(h.astype(jnp.bfloat16), w3_ref[...],
                            preferred_element_type=jnp.float32) + b3_ref[...])
    deep = (jnp.dot(h.astype(jnp.bfloat16), wo_ref[...],
                    preferred_element_type=jnp.float32) + bo_ref[...])
    o_ref[...] = jax.nn.sigmoid(first + second + deep)


def _full(shape):
    return pl.BlockSpec(shape, lambda i: (0,) * len(shape))


_tc_call = pl.pallas_call(
    _tc_body,
    grid=(BATCH // BB,),
    in_specs=[
        pl.BlockSpec((BB, FGROUPS[0] * 128), lambda i: (i, 0)),
        pl.BlockSpec((BB, FGROUPS[1] * 128), lambda i: (i, 0)),
        pl.BlockSpec((BB, N_FIELDS * 16), lambda i: (i, 0)),
        pl.BlockSpec((BB, N_FIELDS), lambda i: (i, 0)),
        _full((DA, HID)),
        _full((DA, HID)),
        _full((1, HID)),
        _full((HID, HID)),
        _full((1, HID)),
        _full((HID, HID)),
        _full((1, HID)),
        _full((HID, 1)),
        _full((1, 1)),
    ],
    out_specs=pl.BlockSpec((BB, 1), lambda i: (i, 0)),
    out_shape=jax.ShapeDtypeStruct((BATCH, 1), jnp.float32),
)


def kernel(sparse_inputs, tables, w, W1, b1, W2, b2, W3, b3, Wout, bout):
    tt = jnp.transpose(tables, (0, 2, 1)).reshape(N_FIELDS * EMBED, VOCAB)

    offs = (jnp.arange(N_FIELDS, dtype=jnp.int32) * VOCAB)[None, :]
    idx = (sparse_inputs.astype(jnp.int32) + offs)
    w16 = w.reshape(N_FIELDS * VOCAB // 16, 16)
    w16rows = _make_sc_wgather()((idx >> 4).reshape(NW, CHUNKS_PER_W, CHUNK), w16)

    xps = []
    f0 = 0
    for nf in FGROUPS:
        tabp = _make_tr_call(f0, nf)(tt).reshape(nf * VOCAB, 128)
        g_rows = BATCH * nf
        idxh = (idx[:, f0:f0 + nf] - f0 * VOCAB)
        idxh = idxh.reshape(NW, g_rows // (NW * CHUNK), CHUNK)
        xps.append(_make_sc_gather(g_rows)(idxh, tabp).reshape(BATCH, nf * 128))
        f0 += nf

    wv = w16rows.reshape(BATCH, N_FIELDS * 16)
    lo = (idx & 15).reshape(BATCH, N_FIELDS)

    w1s = W1.reshape(N_FIELDS, EMBED, HID)
    w1a = w1s[:, :128, :].reshape(DA, HID).astype(jnp.bfloat16)
    w1b = jnp.pad(w1s[:, 128:, :], ((0, 0), (0, 128 - (EMBED - 128)), (0, 0))
                  ).reshape(DA, HID).astype(jnp.bfloat16)

    return _tc_call(xps[0], xps[1], wv, lo, w1a, w1b,
                    b1.reshape(1, HID), W2.astype(jnp.bfloat16),
                    b2.reshape(1, HID), W3.astype(jnp.bfloat16),
                    b3.reshape(1, HID), Wout.astype(jnp.bfloat16),
                    bout.reshape(1, 1))
